# Initial kernel scaffold; baseline (speedup 1.0000x reference)
import jax
import jax.numpy as jnp
from jax.experimental import pallas as pl


def kernel(z, v_in, g_in, b_in, v_out, g_out, b_out, codebook):
    raise NotImplementedError("write your pallas kernel here")


# speedup vs baseline: 1.1071x; 1.1071x over previous
import functools

import jax
import jax.numpy as jnp
from jax import lax
from jax.experimental import pallas as pl
from jax.experimental.pallas import tpu as pltpu, tpu_sc as plsc

B, D, T = 8, 1024, 2048
CB_DIM, CB_SIZE = 8, 8192
TB = 256
DPAD = 128


def _row_sum8(sq):
    s1 = sq[:, 0:4] + sq[:, 4:8]
    s2 = s1[:, 0:2] + s1[:, 2:4]
    return s2[:, 0:1] + s2[:, 1:2]


def _encode_body(z_ref, w_in_ref, b_in_ref, cbnt_ref, c_ref, idx_ref, ze_ref):
    zb = z_ref[0]
    ze = jnp.dot(w_in_ref[...], zb, preferred_element_type=jnp.float32)
    ze = ze + b_in_ref[...]
    ze_ref[0] = ze

    zet = ze.T
    n2 = _row_sum8(zet * zet)
    en = zet / jnp.maximum(jnp.sqrt(n2), 1e-12)
    a = _row_sum8(en * en)

    m = jnp.dot(en, cbnt_ref[...], preferred_element_type=jnp.float32)
    sc = -((a - 2.0 * m) + c_ref[...])

    mx = jnp.max(sc, axis=1, keepdims=True)
    ii = jax.lax.broadcasted_iota(jnp.int32, (TB, CB_SIZE), 1)
    idx = jnp.min(jnp.where(sc == mx, ii, CB_SIZE), axis=1, keepdims=True)
    idx_ref[0] = idx


def _decode_body(q_ref, ze_ref, w_out_ref, b_out_ref, zq_ref):
    q = q_ref[0][:, 0:CB_DIM]
    ze = ze_ref[0]
    qst = ze + (q.T - ze)
    out = jnp.dot(w_out_ref[...], qst, preferred_element_type=jnp.float32)
    zq_ref[0] = out + b_out_ref[...]


def _sc_gather(table, idx_flat):
    info = plsc.get_sparse_core_info()
    nw = info.num_cores * info.num_subcores
    b_per_w = (B * T) // nw
    mesh = plsc.VectorSubcoreMesh(core_axis_name="c", subcore_axis_name="s")

    @functools.partial(
        pl.kernel, mesh=mesh,
        out_type=jax.ShapeDtypeStruct((B * T, DPAD), jnp.float32),
        scratch_types=[
            pltpu.VMEM((b_per_w,), jnp.int32),
            pltpu.VMEM((b_per_w, DPAD), jnp.float32),
            pltpu.SemaphoreType.DMA,
        ],
    )
    def gather_k(table_hbm, idx_hbm, out_hbm, idx_v, rows_v, sem):
        wid = lax.axis_index("s") * info.num_cores + lax.axis_index("c")
        base = wid * b_per_w
        pltpu.sync_copy(idx_hbm.at[pl.ds(base, b_per_w)], idx_v)
        pltpu.async_copy(table_hbm.at[idx_v], rows_v, sem).wait()
        pltpu.sync_copy(rows_v, out_hbm.at[pl.ds(base, b_per_w)])

    return gather_k(table, idx_flat)


@functools.partial(jax.jit, static_argnames=())
def kernel(z, v_in, g_in, b_in, v_out, g_out, b_out, codebook):
    n_in = jnp.sqrt(jnp.sum(v_in * v_in, axis=(1, 2), keepdims=True))
    w_in = (g_in * v_in / n_in)[:, :, 0]
    n_out = jnp.sqrt(jnp.sum(v_out * v_out, axis=(1, 2), keepdims=True))
    w_out = (g_out * v_out / n_out)[:, :, 0]
    cb_n = codebook / jnp.maximum(
        jnp.linalg.norm(codebook, axis=1, keepdims=True), 1e-12)
    c_row = jnp.sum(cb_n * cb_n, axis=1)[None, :]
    cb_pad = jnp.pad(codebook, ((0, 0), (0, DPAD - CB_DIM)))

    grid = (B, T // TB)
    idx3, ze = pl.pallas_call(
        _encode_body,
        grid=grid,
        in_specs=[
            pl.BlockSpec((1, D, TB), lambda b, t: (b, 0, t)),
            pl.BlockSpec((CB_DIM, D), lambda b, t: (0, 0)),
            pl.BlockSpec((CB_DIM, 1), lambda b, t: (0, 0)),
            pl.BlockSpec((CB_DIM, CB_SIZE), lambda b, t: (0, 0)),
            pl.BlockSpec((1, CB_SIZE), lambda b, t: (0, 0)),
        ],
        out_specs=[
            pl.BlockSpec((1, TB, 1), lambda b, t: (b, t, 0)),
            pl.BlockSpec((1, CB_DIM, TB), lambda b, t: (b, 0, t)),
        ],
        out_shape=[
            jax.ShapeDtypeStruct((B, T, 1), jnp.int32),
            jax.ShapeDtypeStruct((B, CB_DIM, T), jnp.float32),
        ],
    )(z, w_in, b_in[:, None], cb_n.T, c_row)

    q_rows = _sc_gather(cb_pad, idx3.reshape(B * T))
    q_rows = q_rows.reshape(B, T, DPAD)

    zq = pl.pallas_call(
        _decode_body,
        grid=grid,
        in_specs=[
            pl.BlockSpec((1, TB, DPAD), lambda b, t: (b, t, 0)),
            pl.BlockSpec((1, CB_DIM, TB), lambda b, t: (b, 0, t)),
            pl.BlockSpec((D, CB_DIM), lambda b, t: (0, 0)),
            pl.BlockSpec((D, 1), lambda b, t: (0, 0)),
        ],
        out_specs=pl.BlockSpec((1, D, TB), lambda b, t: (b, 0, t)),
        out_shape=jax.ShapeDtypeStruct((B, D, T), jnp.float32),
    )(q_rows, ze, w_out, b_out[:, None])

    return zq, idx3.reshape(B, T), ze
